# speedup vs baseline: 4.7241x; 4.7241x over previous
import functools

import jax
import jax.numpy as jnp
from jax import lax
from jax.experimental import pallas as pl
from jax.experimental.pallas import tpu as pltpu
from jax.experimental.pallas import tpu_sc as plsc

NUM_CORES = 2
NUM_SUBCORES = 16
NUM_WORKERS = NUM_CORES * NUM_SUBCORES
CHUNK = 128


def kernel(indices, table):
    b, s = indices.shape
    v, d = table.shape
    n = b * s
    scale = float(d) ** 0.5

    rows_per_worker = n // NUM_WORKERS
    n_chunks = rows_per_worker // CHUNK

    idx_flat = indices.reshape(NUM_WORKERS, n_chunks, CHUNK).astype(jnp.int32)

    mesh = plsc.VectorSubcoreMesh(core_axis_name="c", subcore_axis_name="s")

    @functools.partial(
        pl.kernel,
        mesh=mesh,
        out_type=jax.ShapeDtypeStruct((n, d), jnp.float32),
        scratch_types=[
            pltpu.VMEM((n_chunks, CHUNK), jnp.int32),
            pltpu.VMEM((CHUNK, d), jnp.float32),
            pltpu.SemaphoreType.DMA,
        ],
    )
    def emb_kernel(idx_hbm, table_hbm, out_hbm, idx_v, rows_v, sem):
        wid = lax.axis_index("s") * NUM_CORES + lax.axis_index("c")
        base = wid * rows_per_worker
        pltpu.sync_copy(idx_hbm.at[wid], idx_v)

        def chunk_body(j, _):
            pltpu.async_copy(table_hbm.at[idx_v.at[j]], rows_v, sem).wait()

            def scale_row(r, _):
                for c in range(d // 16):
                    sl = pl.ds(c * 16, 16)
                    rows_v[r, sl] = rows_v[r, sl] * scale
                return ()

            lax.fori_loop(0, CHUNK, scale_row, ())
            pltpu.sync_copy(rows_v, out_hbm.at[pl.ds(base + j * CHUNK, CHUNK)])
            return ()

        lax.fori_loop(0, n_chunks, chunk_body, ())

    out = emb_kernel(idx_flat, table)
    return out.reshape(b, s, d)


# speedup vs baseline: 7.7578x; 1.6422x over previous
import functools

import jax
import jax.numpy as jnp
from jax import lax
from jax.experimental import pallas as pl
from jax.experimental.pallas import tpu as pltpu
from jax.experimental.pallas import tpu_sc as plsc

NUM_CORES = 2
NUM_SUBCORES = 16
NUM_WORKERS = NUM_CORES * NUM_SUBCORES
CHUNK = 128
ROWS_PER_ITER = 8


def kernel(indices, table):
    b_, s_ = indices.shape
    v, d = table.shape
    n = b_ * s_
    scale = float(d) ** 0.5

    rows_per_worker = n // NUM_WORKERS
    n_chunks = rows_per_worker // CHUNK

    idx_flat = indices.reshape(NUM_WORKERS, n_chunks, CHUNK).astype(jnp.int32)

    mesh = plsc.VectorSubcoreMesh(core_axis_name="c", subcore_axis_name="s")

    @functools.partial(
        pl.kernel,
        mesh=mesh,
        out_type=jax.ShapeDtypeStruct((n, d), jnp.float32),
        scratch_types=[
            pltpu.VMEM((n_chunks, CHUNK), jnp.int32),
            pltpu.VMEM((CHUNK, d), jnp.float32),
            pltpu.VMEM((CHUNK, d), jnp.float32),
            pltpu.VMEM((CHUNK, d), jnp.float32),
            pltpu.VMEM((CHUNK, d), jnp.float32),
            pltpu.SemaphoreType.DMA,
            pltpu.SemaphoreType.DMA,
            pltpu.SemaphoreType.DMA,
            pltpu.SemaphoreType.DMA,
        ],
    )
    def emb_kernel(idx_hbm, table_hbm, out_hbm, idx_v, g0, g1, s0, s1,
                   gsem0, gsem1, ssem0, ssem1):
        G = [g0, g1]
        S = [s0, s1]
        GS = [gsem0, gsem1]
        SS = [ssem0, ssem1]

        wid = lax.axis_index("s") * NUM_CORES + lax.axis_index("c")
        base = wid * rows_per_worker
        pltpu.sync_copy(idx_hbm.at[wid], idx_v)

        def issue_gather(j, b):
            pltpu.async_copy(table_hbm.at[idx_v.at[j]], G[b], GS[b])

        def wait_gather(b):
            pltpu.make_async_copy(table_hbm.at[pl.ds(0, CHUNK)], G[b],
                                  GS[b]).wait()

        def issue_scatter(j, b):
            pltpu.async_copy(S[b], out_hbm.at[pl.ds(base + j * CHUNK, CHUNK)],
                             SS[b])

        def wait_scatter(b):
            pltpu.make_async_copy(table_hbm.at[pl.ds(0, CHUNK)], S[b],
                                  SS[b]).wait()

        def scale_chunk(gbuf, sbuf):
            def body(i, _):
                r = i * ROWS_PER_ITER
                for rr in range(ROWS_PER_ITER):
                    for c in range(d // 16):
                        sl = pl.ds(c * 16, 16)
                        sbuf[r + rr, sl] = gbuf[r + rr, sl] * scale
                return ()

            lax.fori_loop(0, CHUNK // ROWS_PER_ITER, body, ())

        for b in range(2):
            issue_gather(b, b)
        for b in range(2):
            wait_gather(b)
            scale_chunk(G[b], S[b])
            issue_gather(2 + b, b)
            issue_scatter(b, b)

        def outer(k, _):
            jj = k * 2
            for b in range(2):
                j = jj + b
                wait_gather(b)
                wait_scatter(b)
                scale_chunk(G[b], S[b])
                issue_gather(j + 2, b)
                issue_scatter(j, b)
            return ()

        lax.fori_loop(1, n_chunks // 2 - 1, outer, ())

        for b in range(2):
            j = n_chunks - 2 + b
            wait_gather(b)
            wait_scatter(b)
            scale_chunk(G[b], S[b])
            issue_scatter(j, b)
        for b in range(2):
            wait_scatter(b)

    out = emb_kernel(idx_flat, table)
    return out.reshape(b_, s_, d)
